# baseline (speedup 1.0000x reference)
import jax
import jax.numpy as jnp
from jax import lax
from jax.experimental import pallas as pl
from jax.experimental.pallas import tpu as pltpu
from jax.experimental.pallas import tpu_sc as plsc

NUM_FIELDS = 26
VOCAB = 100000
EMBED_DIM = 32
BATCH = 16384

_info = plsc.get_sparse_core_info()
NC, NS, L = _info.num_cores, _info.num_subcores, _info.num_lanes
NW = NC * NS
B_PER_W = BATCH // NW
CHUNK = 128
N_CHUNK = B_PER_W // CHUNK
IDX_ROWS = NUM_FIELDS * N_CHUNK


def _body(idx_hbm, tab_hbm, *refs):
    outs = refs[:NUM_FIELDS]
    idx_v, rows_a, rows_b, tbuf, sem_a, sem_b, osem = refs[NUM_FIELDS:]
    wid = lax.axis_index("s") * NC + lax.axis_index("c")
    obase = wid * B_PER_W

    pltpu.sync_copy(idx_hbm.at[wid], idx_v)

    rids = [lax.iota(jnp.int32, L) + (i * L) for i in range(B_PER_W // L)]

    def _drain_out():
        pltpu.make_async_copy(
            tab_hbm.at[0].at[pl.ds(0, B_PER_W)], rows_a, osem
        ).wait()

    @pl.loop(0, NUM_FIELDS)
    def _field(ff):
        cps = []
        for j in range(N_CHUNK):
            cps.append(
                pltpu.async_copy(
                    tab_hbm.at[ff].at[idx_v.at[ff * N_CHUNK + j]],
                    rows_a.at[pl.ds(j * CHUNK, CHUNK)],
                    sem_a,
                )
            )
        for c in cps:
            c.wait()
        @pl.when(ff >= 1)
        def _():
            _drain_out()

        for c in range(EMBED_DIM):
            cid = jnp.full((L,), c, jnp.int32)
            for i in range(B_PER_W // L):
                tbuf[c, pl.ds(i * L, L)] = plsc.load_gather(
                    rows_a, [rids[i], cid]
                )
        for k in range(NUM_FIELDS):
            @pl.when(ff == k)
            def _(k=k):
                pltpu.async_copy(
                    tbuf, outs[k].at[:, pl.ds(obase, B_PER_W)], osem
                )

    _drain_out()


@jax.jit
def _encode(idx3d, tables):
    mesh = plsc.VectorSubcoreMesh(core_axis_name="c", subcore_axis_name="s")
    return pl.kernel(
        _body,
        out_type=tuple(
            jax.ShapeDtypeStruct((EMBED_DIM, BATCH), jnp.float32)
            for _ in range(NUM_FIELDS)
        ),
        mesh=mesh,
        scratch_types=[
            pltpu.VMEM((IDX_ROWS, CHUNK), jnp.int32),
            pltpu.VMEM((B_PER_W, EMBED_DIM), jnp.float32),
            pltpu.VMEM((B_PER_W, EMBED_DIM), jnp.float32),
            pltpu.VMEM((EMBED_DIM, B_PER_W), jnp.float32),
            pltpu.SemaphoreType.DMA,
            pltpu.SemaphoreType.DMA,
            pltpu.SemaphoreType.DMA,
        ],
        compiler_params=pltpu.CompilerParams(
            use_tc_tiling_on_sc=False, needs_layout_passes=False
        ),
    )(idx3d, tables)


def kernel(inputs, tables):
    idx3d = (
        inputs.astype(jnp.int32)
        .reshape(NW, B_PER_W, NUM_FIELDS)
        .transpose(0, 2, 1)
        .reshape(NW, IDX_ROWS, CHUNK)
    )
    outs_t = _encode(idx3d, tables)
    return tuple(o.T for o in outs_t)


# speedup vs baseline: 1.1149x; 1.1149x over previous
import jax
import jax.numpy as jnp
from jax import lax
from jax.experimental import pallas as pl
from jax.experimental.pallas import tpu as pltpu
from jax.experimental.pallas import tpu_sc as plsc

NUM_FIELDS = 26
VOCAB = 100000
EMBED_DIM = 32
BATCH = 16384

_info = plsc.get_sparse_core_info()
NC, NS, L = _info.num_cores, _info.num_subcores, _info.num_lanes
NW = NC * NS
B_PER_W = BATCH // NW
CHUNK = 128
N_CHUNK = B_PER_W // CHUNK
IDX_ROWS = NUM_FIELDS * N_CHUNK


def _body(idx_hbm, tab_hbm, *refs):
    outs = refs[:NUM_FIELDS]
    idx_v, rows_a, rows_b, tbuf, sem_a, sem_b, osem = refs[NUM_FIELDS:]
    wid = lax.axis_index("s") * NC + lax.axis_index("c")
    obase = wid * B_PER_W

    pltpu.sync_copy(idx_hbm.at[wid], idx_v)

    bufs = (rows_a, rows_b)
    sems = (sem_a, sem_b)
    gathers = [None, None]

    rids = [lax.iota(jnp.int32, L) + (i * L) for i in range(B_PER_W // L // 2)]

    def _drain_out():
        pltpu.make_async_copy(
            tab_hbm.at[0].at[pl.ds(0, B_PER_W)], bufs[0], osem
        ).wait()

    def _write_out(buf, out):
        @plsc.parallel_loop(0, EMBED_DIM * 2, unroll=4)
        def _col(hc):
            c = hc // 2
            h = (hc & 1) * (B_PER_W // 2)
            cid = jnp.full((L,), 0, jnp.int32) + c
            for i in range(B_PER_W // L // 2):
                tbuf[c, pl.ds(h + i * L, L)] = plsc.load_gather(
                    buf, [rids[i] + h, cid]
                )

        pltpu.async_copy(tbuf, out.at[:, pl.ds(obase, B_PER_W)], osem)

    for f in range(NUM_FIELDS):
        b = f & 1
        if f >= 2:
            _drain_out()
        cps = []
        for j in range(N_CHUNK):
            cps.append(
                pltpu.async_copy(
                    tab_hbm.at[f].at[idx_v.at[f * N_CHUNK + j]],
                    bufs[b].at[pl.ds(j * CHUNK, CHUNK)],
                    sems[b],
                )
            )
        gathers[b] = cps
        if f >= 1:
            pb = 1 - b
            for c in gathers[pb]:
                c.wait()
            _write_out(bufs[pb], outs[f - 1])
    lb = (NUM_FIELDS - 1) & 1
    for c in gathers[lb]:
        c.wait()
    _drain_out()
    _write_out(bufs[lb], outs[NUM_FIELDS - 1])
    _drain_out()


@jax.jit
def _encode(idx3d, tables):
    mesh = plsc.VectorSubcoreMesh(core_axis_name="c", subcore_axis_name="s")
    return pl.kernel(
        _body,
        out_type=tuple(
            jax.ShapeDtypeStruct((EMBED_DIM, BATCH), jnp.float32)
            for _ in range(NUM_FIELDS)
        ),
        mesh=mesh,
        scratch_types=[
            pltpu.VMEM((IDX_ROWS, CHUNK), jnp.int32),
            pltpu.VMEM((B_PER_W, EMBED_DIM), jnp.float32),
            pltpu.VMEM((B_PER_W, EMBED_DIM), jnp.float32),
            pltpu.VMEM((EMBED_DIM, B_PER_W), jnp.float32),
            pltpu.SemaphoreType.DMA,
            pltpu.SemaphoreType.DMA,
            pltpu.SemaphoreType.DMA,
        ],
        compiler_params=pltpu.CompilerParams(
            use_tc_tiling_on_sc=False, needs_layout_passes=False
        ),
    )(idx3d, tables)


def kernel(inputs, tables):
    idx3d = (
        inputs.astype(jnp.int32)
        .reshape(NW, B_PER_W, NUM_FIELDS)
        .transpose(0, 2, 1)
        .reshape(NW, IDX_ROWS, CHUNK)
    )
    outs_t = _encode(idx3d, tables)
    return tuple(o.T for o in outs_t)


# speedup vs baseline: 1.1155x; 1.0005x over previous
import jax
import jax.numpy as jnp
from jax import lax
from jax.experimental import pallas as pl
from jax.experimental.pallas import tpu as pltpu
from jax.experimental.pallas import tpu_sc as plsc

NUM_FIELDS = 26
VOCAB = 100000
EMBED_DIM = 32
BATCH = 16384

_info = plsc.get_sparse_core_info()
NC, NS, L = _info.num_cores, _info.num_subcores, _info.num_lanes
NW = NC * NS
B_PER_W = BATCH // NW
CHUNK = 128
N_CHUNK = B_PER_W // CHUNK
IDX_ROWS = NUM_FIELDS * N_CHUNK


def _body(idx_hbm, tab_hbm, *refs):
    outs = refs[:NUM_FIELDS]
    idx_v, rows_a, rows_b, tbuf, sem_a, sem_b, osem = refs[NUM_FIELDS:]
    wid = lax.axis_index("s") * NC + lax.axis_index("c")
    obase = wid * B_PER_W

    pltpu.sync_copy(idx_hbm.at[wid], idx_v)

    bufs = (rows_a, rows_b)
    sems = (sem_a, sem_b)
    gathers = [None, None]

    cid0 = lax.iota(jnp.int32, L)
    cid1 = cid0 + L

    def _drain_out():
        pltpu.make_async_copy(
            tab_hbm.at[0].at[pl.ds(0, B_PER_W)], bufs[0], osem
        ).wait()

    def _write_out(buf, out):
        @plsc.parallel_loop(0, B_PER_W, unroll=8)
        def _row(j):
            bid = jnp.full((L,), 0, jnp.int32) + j
            plsc.store_scatter(tbuf, [cid0, bid], buf[j, pl.ds(0, L)])
            plsc.store_scatter(tbuf, [cid1, bid], buf[j, pl.ds(L, L)])

        pltpu.async_copy(tbuf, out.at[:, pl.ds(obase, B_PER_W)], osem)

    for f in range(NUM_FIELDS):
        b = f & 1
        if f >= 2:
            _drain_out()
        cps = []
        for j in range(N_CHUNK):
            cps.append(
                pltpu.async_copy(
                    tab_hbm.at[f].at[idx_v.at[f * N_CHUNK + j]],
                    bufs[b].at[pl.ds(j * CHUNK, CHUNK)],
                    sems[b],
                )
            )
        gathers[b] = cps
        if f >= 1:
            pb = 1 - b
            for c in gathers[pb]:
                c.wait()
            _write_out(bufs[pb], outs[f - 1])
    lb = (NUM_FIELDS - 1) & 1
    for c in gathers[lb]:
        c.wait()
    _drain_out()
    _write_out(bufs[lb], outs[NUM_FIELDS - 1])
    _drain_out()


@jax.jit
def _encode(idx3d, tables):
    mesh = plsc.VectorSubcoreMesh(core_axis_name="c", subcore_axis_name="s")
    return pl.kernel(
        _body,
        out_type=tuple(
            jax.ShapeDtypeStruct((EMBED_DIM, BATCH), jnp.float32)
            for _ in range(NUM_FIELDS)
        ),
        mesh=mesh,
        scratch_types=[
            pltpu.VMEM((IDX_ROWS, CHUNK), jnp.int32),
            pltpu.VMEM((B_PER_W, EMBED_DIM), jnp.float32),
            pltpu.VMEM((B_PER_W, EMBED_DIM), jnp.float32),
            pltpu.VMEM((EMBED_DIM, B_PER_W), jnp.float32),
            pltpu.SemaphoreType.DMA,
            pltpu.SemaphoreType.DMA,
            pltpu.SemaphoreType.DMA,
        ],
        compiler_params=pltpu.CompilerParams(
            use_tc_tiling_on_sc=False, needs_layout_passes=False
        ),
    )(idx3d, tables)


def kernel(inputs, tables):
    idx3d = (
        inputs.astype(jnp.int32)
        .reshape(NW, B_PER_W, NUM_FIELDS)
        .transpose(0, 2, 1)
        .reshape(NW, IDX_ROWS, CHUNK)
    )
    outs_t = _encode(idx3d, tables)
    return tuple(o.T for o in outs_t)
